# baseline (speedup 1.0000x reference)
import functools

import jax
import jax.numpy as jnp
from jax import lax
from jax.experimental import pallas as pl
from jax.experimental.pallas import tpu as pltpu
from jax.experimental.pallas import tpu_sc as plsc

B = 1024
P = B * (B - 1) // 2
NW = 32
CHUNK = P // NW
BUFN = CHUNK + 32
XPAD = 1040
TWO_B_M1 = 2 * B - 1


def _mo8(v):
    return pl.multiple_of(v, 8)


def _pairs_body(x_hbm, x1_hbm, x2_hbm, x_v, o1_v, o2_v, sem_x, sem_o):
    wid = lax.axis_index("s") * 2 + lax.axis_index("c")
    cp_x = pltpu.make_async_copy(x_hbm, x_v.at[pl.ds(0, B)], sem_x)
    cp_x.start()
    cp_x.wait()

    def walk_rows(i0, pos0):
        def rbody(r, pos):
            i = i0 + r
            ln = jnp.int32(B - 1) - i
            n = (ln + 15) >> 4
            n8 = n >> 3
            splat = plsc.load_gather(x_v, [jnp.full((16,), i, jnp.int32)])
            j0 = i + 1

            def g8(t, qj):
                q, jq = qj
                for u in range(8):
                    o1_v[pl.ds(q + u * 16, 16)] = splat
                    o2_v[pl.ds(q + u * 16, 16)] = x_v[pl.ds(jq + u * 16, 16)]
                return (q + 128, jq + 128)

            def g4(t, qj):
                q, jq = qj
                for u in range(4):
                    o1_v[pl.ds(q + u * 16, 16)] = splat
                    o2_v[pl.ds(q + u * 16, 16)] = x_v[pl.ds(jq + u * 16, 16)]
                return (q + 64, jq + 64)

            def g1(t, qj):
                q, jq = qj
                o1_v[pl.ds(q, 16)] = splat
                o2_v[pl.ds(q, 16)] = x_v[pl.ds(jq, 16)]
                return (q + 16, jq + 16)

            qj = lax.fori_loop(0, n8, g8, (pos, j0))
            qj = lax.fori_loop(0, (n >> 2) & 1, g4, qj)
            lax.fori_loop(0, n & 3, g1, qj)
            return pos + ln

        return lax.fori_loop(0, 16, rbody, pos0)

    def dma_block(boff, obase, a):
        c2 = a >> 3
        c1 = a & 7

        def start(src_off, dst_off, sz, out_v, out_hbm):
            pltpu.make_async_copy(
                out_v.at[pl.ds(_mo8(src_off), sz)],
                out_hbm.at[pl.ds(_mo8(dst_off), sz)],
                sem_o,
            ).start()

        def d2(t, c):
            s = t * 2048
            start(boff + s, obase + s, 2048, o1_v, x1_hbm)
            start(boff + s, obase + s, 2048, o2_v, x2_hbm)
            return c

        lax.fori_loop(0, c2, d2, 0)
        s1 = c2 << 11

        def d1(t, c):
            s = s1 + t * 256
            start(boff + s, obase + s, 256, o1_v, x1_hbm)
            start(boff + s, obase + s, 256, o2_v, x2_hbm)
            return c

        lax.fori_loop(0, c1, d1, 0)
        st = a << 8
        start(boff + st, obase + st, 120, o1_v, x1_hbm)
        start(boff + st, obase + st, 120, o2_v, x2_hbm)

    iA0 = 16 * wid
    iB0 = jnp.int32(1008) - 16 * wid
    aA = jnp.int32(63) - wid
    la = 16248 - 256 * wid
    oa = (8 * wid) * (TWO_B_M1 - 16 * wid)
    m = jnp.int32(63) - wid
    ob = (8 * m) * (TWO_B_M1 - 16 * m)

    walk_rows(iA0, 0)
    dma_block(0, oa, aA)
    walk_rows(iB0, la + 16)
    dma_block(la + 16, ob, wid)

    pltpu.make_async_copy(
        x1_hbm.at[pl.ds(0, CHUNK)], o1_v.at[pl.ds(0, CHUNK)], sem_o
    ).wait()
    pltpu.make_async_copy(
        x2_hbm.at[pl.ds(0, CHUNK)], o2_v.at[pl.ds(0, CHUNK)], sem_o
    ).wait()


@functools.cache
def _build():
    return functools.partial(
        pl.kernel,
        out_type=(
            jax.ShapeDtypeStruct((P,), jnp.float32),
            jax.ShapeDtypeStruct((P,), jnp.float32),
        ),
        mesh=plsc.VectorSubcoreMesh(
            core_axis_name="c", subcore_axis_name="s", num_cores=2, num_subcores=16
        ),
        scratch_types=[
            pltpu.VMEM((XPAD,), jnp.float32),
            pltpu.VMEM((BUFN,), jnp.float32),
            pltpu.VMEM((BUFN,), jnp.float32),
            pltpu.SemaphoreType.DMA,
            pltpu.SemaphoreType.DMA,
        ],
        compiler_params=pltpu.CompilerParams(
            needs_layout_passes=False, disable_bounds_checks=True
        ),
    )(_pairs_body)


def kernel(x):
    return _build()(x)
